# baseline (speedup 1.0000x reference)
import functools
import math

import numpy as np
import jax
import jax.numpy as jnp
from jax.experimental import pallas as pl
from jax.experimental.pallas import tpu as pltpu

_EPS = 1e-5


def _round_up(v, m):
    return ((v + m - 1) // m) * m


def _width_matrix_np(n_in, n_out):
    if n_in == 1:
        return np.ones((1, n_out), np.float32)
    src = np.arange(n_out, dtype=np.float64) * (n_in - 1) / (n_out - 1)
    grid = np.arange(n_in, dtype=np.float64)
    m = np.maximum(0.0, 1.0 - np.abs(src[None, :] - grid[:, None]))
    return m.astype(np.float32)


def _height_taps(n_in, n_out):
    taps = []
    for dst in range(n_out):
        if n_in == 1:
            taps.append((0, 0, 1.0, 0.0))
            continue
        src = dst * (n_in - 1) / (n_out - 1)
        i0 = min(int(math.floor(src)), n_in - 1)
        i1 = min(i0 + 1, n_in - 1)
        frac = src - i0
        taps.append((i0, i1, 1.0 - frac, float(frac)))
    return tuple(taps)


def _fwd_kernel(aw_ref, ws_ref, x_ref, s_ref, y_ref, sum_ref, ssq_ref,
                t1_s, img_s, rs_s, *, h_taps, h, h2, w2, cin_p, cout):
    p = h2 * w2
    c2 = 2 * cin_p
    f32 = jnp.float32
    bf16 = jnp.bfloat16

    t1_s[...] = jnp.dot(x_ref[0].astype(bf16), aw_ref[...],
                        preferred_element_type=f32).reshape(cin_p, h, w2)

    for t in range(h2 // 2):
        halves = []
        for hh in (2 * t, 2 * t + 1):
            i0, i1, a0, a1 = h_taps[hh]
            r = a0 * t1_s[:, i0, :]
            if a1 != 0.0:
                r = r + a1 * t1_s[:, i1, :]
            halves.append(r)
        img_s[0:cin_p, 2 * t * w2:(2 * t + 2) * w2] = (
            jnp.concatenate(halves, axis=1).astype(bf16))

    img_s[cin_p:c2, :] = s_ref[0].astype(bf16)

    lin = jax.lax.broadcasted_iota(jnp.int32, (1, p), 1)
    wpos = lin - (lin // w2) * w2
    zero = jnp.zeros((), bf16)
    c_blk = img_s[0:c2, :]
    img_s[c2:2 * c2, :] = jnp.where(wpos >= 1,
                                    jnp.roll(c_blk, 1, axis=1), zero)
    img_s[2 * c2:3 * c2, :] = jnp.where(wpos <= w2 - 2,
                                        jnp.roll(c_blk, -1, axis=1), zero)

    rs_s[...] = jnp.dot(ws_ref[...], img_s[...],
                        preferred_element_type=f32).astype(bf16)

    mid = rs_s[cout:2 * cout, :].astype(f32)
    top = jnp.where(lin >= w2,
                    jnp.roll(rs_s[0:cout, :], w2, axis=1), zero).astype(f32)
    bot = jnp.where(lin < p - w2,
                    jnp.roll(rs_s[2 * cout:3 * cout, :], -w2, axis=1),
                    zero).astype(f32)
    y = jnp.maximum(mid + top + bot, 0.0)

    sum_ref[0] = jnp.sum(y, axis=1, keepdims=True)
    ssq_ref[0] = jnp.sum(y * y, axis=1, keepdims=True)
    y_ref[0] = y.astype(bf16)


def _bn_kernel(gamma_ref, beta_ref, sum_ref, ssq_ref, y_ref, o_ref,
               *, cnt, eps):
    f32 = jnp.float32
    s = jnp.sum(sum_ref[...], axis=0)
    q = jnp.sum(ssq_ref[...], axis=0)
    mean = s * (1.0 / cnt)
    var = jnp.maximum(q * (1.0 / cnt) - mean * mean, 0.0)
    scale = gamma_ref[...] * jax.lax.rsqrt(var + eps)
    shift = beta_ref[...] - mean * scale
    o_ref[0] = y_ref[0].astype(f32) * scale + shift


def kernel(x_nchw, skip_nchw, w_hwio, gamma, beta):
    n, cin, h, w = x_nchw.shape
    _, cin_s, h2, w2 = skip_nchw.shape
    kh, kw, cin2, cout = w_hwio.shape
    assert (h2, w2) == (2 * h, 2 * w) and cin_s == cin and cin2 == 2 * cin
    assert kh == 3 and kw == 3
    p = h2 * w2
    cin_p = _round_up(cin, 8)
    c2 = 2 * cin_p
    f32 = jnp.float32
    bf16 = jnp.bfloat16

    aw = jnp.asarray(_width_matrix_np(w, w2), dtype=bf16)
    h_taps = _height_taps(h, h2)

    xp = x_nchw
    sp = skip_nchw
    wq = w_hwio
    if cin_p != cin:
        xp = jnp.pad(xp, ((0, 0), (0, cin_p - cin), (0, 0), (0, 0)))
        sp = jnp.pad(sp, ((0, 0), (0, cin_p - cin), (0, 0), (0, 0)))
        wq = jnp.concatenate(
            [jnp.pad(w_hwio[:, :, :cin, :],
                     ((0, 0), (0, 0), (0, cin_p - cin), (0, 0))),
             jnp.pad(w_hwio[:, :, cin:, :],
                     ((0, 0), (0, 0), (0, cin_p - cin), (0, 0)))], axis=2)
    x_flat = xp.reshape(n, cin_p * h, w)
    s_flat = sp.reshape(n, cin_p, p)

    ws = jnp.transpose(wq[:, jnp.array([1, 0, 2])], (0, 3, 1, 2))
    ws = ws.reshape(kh * cout, 3 * c2).astype(bf16)

    body = functools.partial(_fwd_kernel, h_taps=h_taps, h=h, h2=h2, w2=w2,
                             cin_p=cin_p, cout=cout)

    y_flat, sums, ssqs = pl.pallas_call(
        body,
        out_shape=(jax.ShapeDtypeStruct((n, cout, p), bf16),
                   jax.ShapeDtypeStruct((n, cout, 1), f32),
                   jax.ShapeDtypeStruct((n, cout, 1), f32)),
        grid=(n,),
        in_specs=[
            pl.BlockSpec((w, w2), lambda i: (0, 0)),
            pl.BlockSpec((kh * cout, 3 * c2), lambda i: (0, 0)),
            pl.BlockSpec((1, cin_p * h, w), lambda i: (i, 0, 0)),
            pl.BlockSpec((1, cin_p, p), lambda i: (i, 0, 0)),
        ],
        out_specs=(
            pl.BlockSpec((1, cout, p), lambda i: (i, 0, 0)),
            pl.BlockSpec((1, cout, 1), lambda i: (i, 0, 0)),
            pl.BlockSpec((1, cout, 1), lambda i: (i, 0, 0)),
        ),
        scratch_shapes=[
            pltpu.VMEM((cin_p, h, w2), f32),
            pltpu.VMEM((3 * c2, p), bf16),
            pltpu.VMEM((kh * cout, p), bf16),
        ],
        compiler_params=pltpu.CompilerParams(
            dimension_semantics=("parallel",)),
    )(aw, ws, x_flat, s_flat)

    bn_body = functools.partial(_bn_kernel, cnt=float(n * p), eps=_EPS)
    out_flat = pl.pallas_call(
        bn_body,
        out_shape=jax.ShapeDtypeStruct((n, cout, p), f32),
        grid=(n,),
        in_specs=[
            pl.BlockSpec((cout, 1), lambda i: (0, 0)),
            pl.BlockSpec((cout, 1), lambda i: (0, 0)),
            pl.BlockSpec((n, cout, 1), lambda i: (0, 0, 0)),
            pl.BlockSpec((n, cout, 1), lambda i: (0, 0, 0)),
            pl.BlockSpec((1, cout, p), lambda i: (i, 0, 0)),
        ],
        out_specs=pl.BlockSpec((1, cout, p), lambda i: (i, 0, 0)),
        compiler_params=pltpu.CompilerParams(
            dimension_semantics=("parallel",)),
    )(gamma.reshape(cout, 1).astype(f32), beta.reshape(cout, 1).astype(f32),
      sums, ssqs, y_flat)

    return out_flat.reshape(n, cout, h2, w2).astype(x_nchw.dtype)


# speedup vs baseline: 1.2731x; 1.2731x over previous
import functools
import math

import numpy as np
import jax
import jax.numpy as jnp
from jax.experimental import pallas as pl
from jax.experimental.pallas import tpu as pltpu

_EPS = 1e-5


def _round_up(v, m):
    return ((v + m - 1) // m) * m


def _width_matrix_np(n_in, n_out):
    if n_in == 1:
        return np.ones((1, n_out), np.float32)
    src = np.arange(n_out, dtype=np.float64) * (n_in - 1) / (n_out - 1)
    grid = np.arange(n_in, dtype=np.float64)
    m = np.maximum(0.0, 1.0 - np.abs(src[None, :] - grid[:, None]))
    return m.astype(np.float32)


def _height_taps(n_in, n_out):
    taps = []
    for dst in range(n_out):
        if n_in == 1:
            taps.append((0, 0, 1.0, 0.0))
            continue
        src = dst * (n_in - 1) / (n_out - 1)
        i0 = min(int(math.floor(src)), n_in - 1)
        i1 = min(i0 + 1, n_in - 1)
        frac = src - i0
        taps.append((i0, i1, 1.0 - frac, float(frac)))
    return tuple(taps)


def _fwd_kernel(aw_ref, w_ref, x_ref, s_ref, y_ref, sum_ref, ssq_ref,
                t1_s, img_s, rs_s, *, h_taps, kw, h2, w2, cin_p, cout):
    p = h2 * w2
    c2 = 2 * cin_p
    f32 = jnp.float32
    bf16 = jnp.bfloat16

    t1_s[...] = jnp.dot(x_ref[0], aw_ref[...], preferred_element_type=f32)

    for t in range(h2 // 2):
        halves = []
        for hh in (2 * t, 2 * t + 1):
            i0, i1, a0, a1 = h_taps[hh]
            r = a0 * t1_s[i0 * cin_p:(i0 + 1) * cin_p, :]
            if a1 != 0.0:
                r = r + a1 * t1_s[i1 * cin_p:(i1 + 1) * cin_p, :]
            halves.append(r)
        img_s[c2:c2 + cin_p, 2 * t * w2:(2 * t + 2) * w2] = (
            jnp.concatenate(halves, axis=1).astype(bf16))

    img_s[c2 + cin_p:2 * c2, :] = s_ref[0].astype(bf16)

    lin = jax.lax.broadcasted_iota(jnp.int32, (1, p), 1)
    wpos = lin - (lin // w2) * w2
    zero = jnp.zeros((), bf16)
    c_blk = img_s[c2:2 * c2, :]
    img_s[0:c2, :] = jnp.where(wpos >= 1, jnp.roll(c_blk, 1, axis=1), zero)
    img_s[2 * c2:3 * c2, :] = jnp.where(wpos <= w2 - 2,
                                        jnp.roll(c_blk, -1, axis=1), zero)

    img = img_s[...]
    for ky in range(3):
        wk = w_ref[ky].reshape(kw * c2, cout).astype(bf16)
        rs_s[ky * cout:(ky + 1) * cout, :] = jax.lax.dot_general(
            wk, img, (((0,), (0,)), ((), ())),
            preferred_element_type=f32).astype(bf16)

    mid = rs_s[cout:2 * cout, :].astype(f32)
    top = jnp.where(lin >= w2,
                    jnp.roll(rs_s[0:cout, :], w2, axis=1), zero).astype(f32)
    bot = jnp.where(lin < p - w2,
                    jnp.roll(rs_s[2 * cout:3 * cout, :], -w2, axis=1),
                    zero).astype(f32)
    y = jnp.maximum(mid + top + bot, 0.0)

    sum_ref[0] = jnp.sum(y, axis=1, keepdims=True)
    ssq_ref[0] = jnp.sum(y * y, axis=1, keepdims=True)
    y_ref[0] = y.astype(bf16)


def _bn_kernel(gamma_ref, beta_ref, sum_ref, ssq_ref, y_ref, o_ref,
               *, cnt, eps):
    f32 = jnp.float32
    s = jnp.sum(sum_ref[...], axis=0)
    q = jnp.sum(ssq_ref[...], axis=0)
    mean = s * (1.0 / cnt)
    var = jnp.maximum(q * (1.0 / cnt) - mean * mean, 0.0)
    scale = gamma_ref[...] * jax.lax.rsqrt(var + eps)
    shift = beta_ref[...] - mean * scale
    o_ref[0] = y_ref[0].astype(f32) * scale + shift


def kernel(x_nchw, skip_nchw, w_hwio, gamma, beta):
    n, cin, h, w = x_nchw.shape
    _, cin_s, h2, w2 = skip_nchw.shape
    kh, kw, cin2, cout = w_hwio.shape
    assert (h2, w2) == (2 * h, 2 * w) and cin_s == cin and cin2 == 2 * cin
    assert kh == 3 and kw == 3
    p = h2 * w2
    cin_p = _round_up(cin, 8)
    c2 = 2 * cin_p
    f32 = jnp.float32
    bf16 = jnp.bfloat16

    aw = jnp.asarray(_width_matrix_np(w, w2), dtype=bf16)
    h_taps = _height_taps(h, h2)

    xp = x_nchw
    sp = skip_nchw
    wq = w_hwio
    if cin_p != cin:
        xp = jnp.pad(xp, ((0, 0), (0, cin_p - cin), (0, 0), (0, 0)))
        sp = jnp.pad(sp, ((0, 0), (0, cin_p - cin), (0, 0), (0, 0)))
        wq = jnp.concatenate(
            [jnp.pad(w_hwio[:, :, :cin, :],
                     ((0, 0), (0, 0), (0, cin_p - cin), (0, 0))),
             jnp.pad(w_hwio[:, :, cin:, :],
                     ((0, 0), (0, 0), (0, cin_p - cin), (0, 0)))], axis=2)
    x2d = jnp.transpose(xp, (0, 2, 1, 3)).reshape(n, h * cin_p, w)
    x2d = x2d.astype(bf16)
    s_flat = sp.reshape(n, cin_p, p)

    body = functools.partial(_fwd_kernel, h_taps=h_taps, kw=kw, h2=h2, w2=w2,
                             cin_p=cin_p, cout=cout)

    y_flat, sums, ssqs = pl.pallas_call(
        body,
        out_shape=(jax.ShapeDtypeStruct((n, cout, p), bf16),
                   jax.ShapeDtypeStruct((n, cout, 1), f32),
                   jax.ShapeDtypeStruct((n, cout, 1), f32)),
        grid=(n,),
        in_specs=[
            pl.BlockSpec((w, w2), lambda i: (0, 0)),
            pl.BlockSpec((kh, kw, c2, cout), lambda i: (0, 0, 0, 0)),
            pl.BlockSpec((1, h * cin_p, w), lambda i: (i, 0, 0)),
            pl.BlockSpec((1, cin_p, p), lambda i: (i, 0, 0)),
        ],
        out_specs=(
            pl.BlockSpec((1, cout, p), lambda i: (i, 0, 0)),
            pl.BlockSpec((1, cout, 1), lambda i: (i, 0, 0)),
            pl.BlockSpec((1, cout, 1), lambda i: (i, 0, 0)),
        ),
        scratch_shapes=[
            pltpu.VMEM((h * cin_p, w2), f32),
            pltpu.VMEM((3 * c2, p), bf16),
            pltpu.VMEM((kh * cout, p), bf16),
        ],
        compiler_params=pltpu.CompilerParams(
            dimension_semantics=("parallel",)),
    )(aw, wq, x2d, s_flat)

    bn_body = functools.partial(_bn_kernel, cnt=float(n * p), eps=_EPS)
    out_flat = pl.pallas_call(
        bn_body,
        out_shape=jax.ShapeDtypeStruct((n, cout, p), f32),
        grid=(n,),
        in_specs=[
            pl.BlockSpec((cout, 1), lambda i: (0, 0)),
            pl.BlockSpec((cout, 1), lambda i: (0, 0)),
            pl.BlockSpec((n, cout, 1), lambda i: (0, 0, 0)),
            pl.BlockSpec((n, cout, 1), lambda i: (0, 0, 0)),
            pl.BlockSpec((1, cout, p), lambda i: (i, 0, 0)),
        ],
        out_specs=pl.BlockSpec((1, cout, p), lambda i: (i, 0, 0)),
        compiler_params=pltpu.CompilerParams(
            dimension_semantics=("parallel",)),
    )(gamma.reshape(cout, 1).astype(f32), beta.reshape(cout, 1).astype(f32),
      sums, ssqs, y_flat)

    return out_flat.reshape(n, cout, h2, w2).astype(x_nchw.dtype)


# speedup vs baseline: 1.3413x; 1.0535x over previous
import functools
import math

import numpy as np
import jax
import jax.numpy as jnp
from jax.experimental import pallas as pl
from jax.experimental.pallas import tpu as pltpu

_EPS = 1e-5


def _round_up(v, m):
    return ((v + m - 1) // m) * m


def _width_matrix_np(n_in, n_out):
    if n_in == 1:
        return np.ones((1, n_out), np.float32)
    src = np.arange(n_out, dtype=np.float64) * (n_in - 1) / (n_out - 1)
    grid = np.arange(n_in, dtype=np.float64)
    m = np.maximum(0.0, 1.0 - np.abs(src[None, :] - grid[:, None]))
    return m.astype(np.float32)


def _height_taps(n_in, n_out):
    taps = []
    for dst in range(n_out):
        if n_in == 1:
            taps.append((0, 0, 1.0, 0.0))
            continue
        src = dst * (n_in - 1) / (n_out - 1)
        i0 = min(int(math.floor(src)), n_in - 1)
        i1 = min(i0 + 1, n_in - 1)
        frac = src - i0
        taps.append((i0, i1, 1.0 - frac, float(frac)))
    return tuple(taps)


def _fwd_kernel(aw_ref, w_ref, x_ref, s_ref, y_ref, sum_ref, ssq_ref,
                t1_s, img_s, rs_s, *, h_taps, kw, h2, w2, cin_p, cout, nb):
    p = h2 * w2
    c2 = 2 * cin_p
    f32 = jnp.float32
    bf16 = jnp.bfloat16

    lin = jax.lax.broadcasted_iota(jnp.int32, (1, p), 1)
    wpos = lin - (lin // w2) * w2
    zero = jnp.zeros((), bf16)

    for bb in range(nb):
        t1_s[bb] = jnp.dot(x_ref[bb], aw_ref[...],
                           preferred_element_type=f32)

        for t in range(h2 // 2):
            halves = []
            for hh in (2 * t, 2 * t + 1):
                i0, i1, a0, a1 = h_taps[hh]
                r = a0 * t1_s[bb, i0 * cin_p:(i0 + 1) * cin_p, :]
                if a1 != 0.0:
                    r = r + a1 * t1_s[bb, i1 * cin_p:(i1 + 1) * cin_p, :]
                halves.append(r)
            img_s[bb, c2:c2 + cin_p, 2 * t * w2:(2 * t + 2) * w2] = (
                jnp.concatenate(halves, axis=1).astype(bf16))

        img_s[bb, c2 + cin_p:2 * c2, :] = s_ref[bb].astype(bf16)

        c_blk = img_s[bb, c2:2 * c2, :]
        img_s[bb, 0:c2, :] = jnp.where(wpos >= 1,
                                       jnp.roll(c_blk, 1, axis=1), zero)
        img_s[bb, 2 * c2:3 * c2, :] = jnp.where(wpos <= w2 - 2,
                                                jnp.roll(c_blk, -1, axis=1),
                                                zero)

        img = img_s[bb]
        for ky in range(3):
            wk = w_ref[ky].reshape(kw * c2, cout).astype(bf16)
            rs_s[bb, ky * cout:(ky + 1) * cout, :] = jax.lax.dot_general(
                wk, img, (((0,), (0,)), ((), ())),
                preferred_element_type=f32).astype(bf16)

        mid = rs_s[bb, cout:2 * cout, :].astype(f32)
        top = jnp.where(lin >= w2,
                        jnp.roll(rs_s[bb, 0:cout, :], w2, axis=1),
                        zero).astype(f32)
        bot = jnp.where(lin < p - w2,
                        jnp.roll(rs_s[bb, 2 * cout:3 * cout, :], -w2, axis=1),
                        zero).astype(f32)
        y = jnp.maximum(mid + top + bot, 0.0)

        sum_ref[bb] = jnp.sum(y, axis=1, keepdims=True)
        ssq_ref[bb] = jnp.sum(y * y, axis=1, keepdims=True)
        y_ref[bb] = y.astype(bf16)


def _bn_kernel(gamma_ref, beta_ref, sum_ref, ssq_ref, y_ref, o_ref,
               *, cnt, eps, nb):
    f32 = jnp.float32
    s = jnp.sum(sum_ref[...], axis=0)
    q = jnp.sum(ssq_ref[...], axis=0)
    mean = s * (1.0 / cnt)
    var = jnp.maximum(q * (1.0 / cnt) - mean * mean, 0.0)
    scale = gamma_ref[...] * jax.lax.rsqrt(var + eps)
    shift = beta_ref[...] - mean * scale
    for bb in range(nb):
        o_ref[bb] = y_ref[bb].astype(f32) * scale + shift


def kernel(x_nchw, skip_nchw, w_hwio, gamma, beta):
    n, cin, h, w = x_nchw.shape
    _, cin_s, h2, w2 = skip_nchw.shape
    kh, kw, cin2, cout = w_hwio.shape
    assert (h2, w2) == (2 * h, 2 * w) and cin_s == cin and cin2 == 2 * cin
    assert kh == 3 and kw == 3
    p = h2 * w2
    cin_p = _round_up(cin, 8)
    c2 = 2 * cin_p
    f32 = jnp.float32
    bf16 = jnp.bfloat16

    aw = jnp.asarray(_width_matrix_np(w, w2), dtype=bf16)
    h_taps = _height_taps(h, h2)

    xp = x_nchw
    sp = skip_nchw
    wq = w_hwio
    if cin_p != cin:
        xp = jnp.pad(xp, ((0, 0), (0, cin_p - cin), (0, 0), (0, 0)))
        sp = jnp.pad(sp, ((0, 0), (0, cin_p - cin), (0, 0), (0, 0)))
        wq = jnp.concatenate(
            [jnp.pad(w_hwio[:, :, :cin, :],
                     ((0, 0), (0, 0), (0, cin_p - cin), (0, 0))),
             jnp.pad(w_hwio[:, :, cin:, :],
                     ((0, 0), (0, 0), (0, cin_p - cin), (0, 0)))], axis=2)
    x2d = jnp.transpose(xp, (0, 2, 1, 3)).reshape(n, h * cin_p, w)
    x2d = x2d.astype(bf16)
    s_flat = sp.reshape(n, cin_p, p)

    nb = 2 if n % 2 == 0 else 1
    body = functools.partial(_fwd_kernel, h_taps=h_taps, kw=kw, h2=h2, w2=w2,
                             cin_p=cin_p, cout=cout, nb=nb)

    y_flat, sums, ssqs = pl.pallas_call(
        body,
        out_shape=(jax.ShapeDtypeStruct((n, cout, p), bf16),
                   jax.ShapeDtypeStruct((n, cout, 1), f32),
                   jax.ShapeDtypeStruct((n, cout, 1), f32)),
        grid=(n // nb,),
        in_specs=[
            pl.BlockSpec((w, w2), lambda i: (0, 0)),
            pl.BlockSpec((kh, kw, c2, cout), lambda i: (0, 0, 0, 0)),
            pl.BlockSpec((nb, h * cin_p, w), lambda i: (i, 0, 0)),
            pl.BlockSpec((nb, cin_p, p), lambda i: (i, 0, 0)),
        ],
        out_specs=(
            pl.BlockSpec((nb, cout, p), lambda i: (i, 0, 0)),
            pl.BlockSpec((nb, cout, 1), lambda i: (i, 0, 0)),
            pl.BlockSpec((nb, cout, 1), lambda i: (i, 0, 0)),
        ),
        scratch_shapes=[
            pltpu.VMEM((nb, h * cin_p, w2), f32),
            pltpu.VMEM((nb, 3 * c2, p), bf16),
            pltpu.VMEM((nb, kh * cout, p), bf16),
        ],
        compiler_params=pltpu.CompilerParams(
            dimension_semantics=("parallel",)),
    )(aw, wq, x2d, s_flat)

    bn_body = functools.partial(_bn_kernel, cnt=float(n * p), eps=_EPS,
                                nb=nb)
    out_flat = pl.pallas_call(
        bn_body,
        out_shape=jax.ShapeDtypeStruct((n, cout, p), f32),
        grid=(n // nb,),
        in_specs=[
            pl.BlockSpec((cout, 1), lambda i: (0, 0)),
            pl.BlockSpec((cout, 1), lambda i: (0, 0)),
            pl.BlockSpec((n, cout, 1), lambda i: (0, 0, 0)),
            pl.BlockSpec((n, cout, 1), lambda i: (0, 0, 0)),
            pl.BlockSpec((nb, cout, p), lambda i: (i, 0, 0)),
        ],
        out_specs=pl.BlockSpec((nb, cout, p), lambda i: (i, 0, 0)),
        compiler_params=pltpu.CompilerParams(
            dimension_semantics=("parallel",)),
    )(gamma.reshape(cout, 1).astype(f32), beta.reshape(cout, 1).astype(f32),
      sums, ssqs, y_flat)

    return out_flat.reshape(n, cout, h2, w2).astype(x_nchw.dtype)


# speedup vs baseline: 1.4001x; 1.0439x over previous
import functools
import math

import numpy as np
import jax
import jax.numpy as jnp
from jax.experimental import pallas as pl
from jax.experimental.pallas import tpu as pltpu

_EPS = 1e-5


def _round_up(v, m):
    return ((v + m - 1) // m) * m


def _width_matrix_np(n_in, n_out):
    if n_in == 1:
        return np.ones((1, n_out), np.float32)
    src = np.arange(n_out, dtype=np.float64) * (n_in - 1) / (n_out - 1)
    grid = np.arange(n_in, dtype=np.float64)
    m = np.maximum(0.0, 1.0 - np.abs(src[None, :] - grid[:, None]))
    return m.astype(np.float32)


def _height_taps(n_in, n_out):
    taps = []
    for dst in range(n_out):
        if n_in == 1:
            taps.append((0, 0, 1.0, 0.0))
            continue
        src = dst * (n_in - 1) / (n_out - 1)
        i0 = min(int(math.floor(src)), n_in - 1)
        i1 = min(i0 + 1, n_in - 1)
        frac = src - i0
        taps.append((i0, i1, 1.0 - frac, float(frac)))
    return tuple(taps)


def _fwd_kernel(aw_ref, w_ref, x_ref, s_ref, y_ref, sum_ref, ssq_ref,
                t1_s, img_s, rs_s, *, h_taps, kw, h2, w2, cin_p, cout, nb):
    p = h2 * w2
    c2 = 2 * cin_p
    f32 = jnp.float32
    bf16 = jnp.bfloat16

    lin = jax.lax.broadcasted_iota(jnp.int32, (1, p), 1)
    wpos = lin - (lin // w2) * w2
    zero = jnp.zeros((), bf16)

    for bb in range(nb):
        t1_s[bb] = jnp.dot(x_ref[bb], aw_ref[...],
                           preferred_element_type=f32)

        for t in range(h2 // 2):
            halves = []
            for hh in (2 * t, 2 * t + 1):
                i0, i1, a0, a1 = h_taps[hh]
                r = a0 * t1_s[bb, i0 * cin_p:(i0 + 1) * cin_p, :]
                if a1 != 0.0:
                    r = r + a1 * t1_s[bb, i1 * cin_p:(i1 + 1) * cin_p, :]
                halves.append(r)
            img_s[bb, c2:c2 + cin_p, 2 * t * w2:(2 * t + 2) * w2] = (
                jnp.concatenate(halves, axis=1).astype(bf16))

        img_s[bb, c2 + cin_p:2 * c2, :] = s_ref[bb].astype(bf16)

        c_blk = img_s[bb, c2:2 * c2, :]
        img_s[bb, 0:c2, :] = jnp.where(wpos >= 1,
                                       jnp.roll(c_blk, 1, axis=1), zero)
        img_s[bb, 2 * c2:3 * c2, :] = jnp.where(wpos <= w2 - 2,
                                                jnp.roll(c_blk, -1, axis=1),
                                                zero)

        img = img_s[bb]
        for ky in range(3):
            wk = w_ref[ky].reshape(kw * c2, cout).astype(bf16)
            rs_s[bb, ky * cout:(ky + 1) * cout, :] = jax.lax.dot_general(
                wk, img, (((0,), (0,)), ((), ())),
                preferred_element_type=f32).astype(bf16)

        mid = rs_s[bb, cout:2 * cout, :].astype(f32)
        top = jnp.where(lin >= w2,
                        jnp.roll(rs_s[bb, 0:cout, :], w2, axis=1),
                        zero).astype(f32)
        bot = jnp.where(lin < p - w2,
                        jnp.roll(rs_s[bb, 2 * cout:3 * cout, :], -w2, axis=1),
                        zero).astype(f32)
        y = jnp.maximum(mid + top + bot, 0.0)

        sum_ref[bb] = jnp.sum(y, axis=1, keepdims=True)
        ssq_ref[bb] = jnp.sum(y * y, axis=1, keepdims=True)
        y_ref[bb] = y.astype(bf16)


def _bn_kernel(gamma_ref, beta_ref, sum_ref, ssq_ref, y_ref, o_ref,
               *, cnt, eps, nb):
    f32 = jnp.float32
    s = jnp.sum(sum_ref[...], axis=0)
    q = jnp.sum(ssq_ref[...], axis=0)
    mean = s * (1.0 / cnt)
    var = jnp.maximum(q * (1.0 / cnt) - mean * mean, 0.0)
    scale = gamma_ref[...] * jax.lax.rsqrt(var + eps)
    shift = beta_ref[...] - mean * scale
    for bb in range(nb):
        o_ref[bb] = y_ref[bb].astype(f32) * scale + shift


def kernel(x_nchw, skip_nchw, w_hwio, gamma, beta):
    n, cin, h, w = x_nchw.shape
    _, cin_s, h2, w2 = skip_nchw.shape
    kh, kw, cin2, cout = w_hwio.shape
    assert (h2, w2) == (2 * h, 2 * w) and cin_s == cin and cin2 == 2 * cin
    assert kh == 3 and kw == 3
    p = h2 * w2
    cin_p = _round_up(cin, 8)
    c2 = 2 * cin_p
    f32 = jnp.float32
    bf16 = jnp.bfloat16

    aw = jnp.asarray(_width_matrix_np(w, w2), dtype=bf16)
    h_taps = _height_taps(h, h2)

    xp = x_nchw
    sp = skip_nchw
    wq = w_hwio
    if cin_p != cin:
        xp = jnp.pad(xp, ((0, 0), (0, cin_p - cin), (0, 0), (0, 0)))
        sp = jnp.pad(sp, ((0, 0), (0, cin_p - cin), (0, 0), (0, 0)))
        wq = jnp.concatenate(
            [jnp.pad(w_hwio[:, :, :cin, :],
                     ((0, 0), (0, 0), (0, cin_p - cin), (0, 0))),
             jnp.pad(w_hwio[:, :, cin:, :],
                     ((0, 0), (0, 0), (0, cin_p - cin), (0, 0)))], axis=2)
    x2d = jnp.transpose(xp, (0, 2, 1, 3)).reshape(n, h * cin_p, w)
    x2d = x2d.astype(bf16)
    s_flat = sp.reshape(n, cin_p, p)

    nb = 4 if n % 4 == 0 else (2 if n % 2 == 0 else 1)
    body = functools.partial(_fwd_kernel, h_taps=h_taps, kw=kw, h2=h2, w2=w2,
                             cin_p=cin_p, cout=cout, nb=nb)

    y_flat, sums, ssqs = pl.pallas_call(
        body,
        out_shape=(jax.ShapeDtypeStruct((n, cout, p), bf16),
                   jax.ShapeDtypeStruct((n, cout, 1), f32),
                   jax.ShapeDtypeStruct((n, cout, 1), f32)),
        grid=(n // nb,),
        in_specs=[
            pl.BlockSpec((w, w2), lambda i: (0, 0)),
            pl.BlockSpec((kh, kw, c2, cout), lambda i: (0, 0, 0, 0)),
            pl.BlockSpec((nb, h * cin_p, w), lambda i: (i, 0, 0)),
            pl.BlockSpec((nb, cin_p, p), lambda i: (i, 0, 0)),
        ],
        out_specs=(
            pl.BlockSpec((nb, cout, p), lambda i: (i, 0, 0)),
            pl.BlockSpec((nb, cout, 1), lambda i: (i, 0, 0)),
            pl.BlockSpec((nb, cout, 1), lambda i: (i, 0, 0)),
        ),
        scratch_shapes=[
            pltpu.VMEM((nb, h * cin_p, w2), f32),
            pltpu.VMEM((nb, 3 * c2, p), bf16),
            pltpu.VMEM((nb, kh * cout, p), bf16),
        ],
        compiler_params=pltpu.CompilerParams(
            dimension_semantics=("parallel",)),
    )(aw, wq, x2d, s_flat)

    bn_body = functools.partial(_bn_kernel, cnt=float(n * p), eps=_EPS,
                                nb=nb)
    out_flat = pl.pallas_call(
        bn_body,
        out_shape=jax.ShapeDtypeStruct((n, cout, p), f32),
        grid=(n // nb,),
        in_specs=[
            pl.BlockSpec((cout, 1), lambda i: (0, 0)),
            pl.BlockSpec((cout, 1), lambda i: (0, 0)),
            pl.BlockSpec((n, cout, 1), lambda i: (0, 0, 0)),
            pl.BlockSpec((n, cout, 1), lambda i: (0, 0, 0)),
            pl.BlockSpec((nb, cout, p), lambda i: (i, 0, 0)),
        ],
        out_specs=pl.BlockSpec((nb, cout, p), lambda i: (i, 0, 0)),
        compiler_params=pltpu.CompilerParams(
            dimension_semantics=("parallel",)),
    )(gamma.reshape(cout, 1).astype(f32), beta.reshape(cout, 1).astype(f32),
      sums, ssqs, y_flat)

    return out_flat.reshape(n, cout, h2, w2).astype(x_nchw.dtype)


# speedup vs baseline: 1.4303x; 1.0216x over previous
import functools
import math

import numpy as np
import jax
import jax.numpy as jnp
from jax.experimental import pallas as pl
from jax.experimental.pallas import tpu as pltpu

_EPS = 1e-5


def _round_up(v, m):
    return ((v + m - 1) // m) * m


def _width_matrix_np(n_in, n_out):
    if n_in == 1:
        return np.ones((1, n_out), np.float32)
    src = np.arange(n_out, dtype=np.float64) * (n_in - 1) / (n_out - 1)
    grid = np.arange(n_in, dtype=np.float64)
    m = np.maximum(0.0, 1.0 - np.abs(src[None, :] - grid[:, None]))
    return m.astype(np.float32)


def _height_taps(n_in, n_out):
    taps = []
    for dst in range(n_out):
        if n_in == 1:
            taps.append((0, 0, 1.0, 0.0))
            continue
        src = dst * (n_in - 1) / (n_out - 1)
        i0 = min(int(math.floor(src)), n_in - 1)
        i1 = min(i0 + 1, n_in - 1)
        frac = src - i0
        taps.append((i0, i1, 1.0 - frac, float(frac)))
    return tuple(taps)


def _fused_kernel(aw_ref, w_ref, g_ref, b_ref, x_ref, s_ref, o_ref,
                  t1_s, img_s, rs_s, y_s, sum_s, ssq_s,
                  *, h_taps, kw, h2, w2, cin_p, cout, nb, nsteps, cnt, eps):
    p = h2 * w2
    c2 = 2 * cin_p
    f32 = jnp.float32
    bf16 = jnp.bfloat16
    i = pl.program_id(0)

    lin = jax.lax.broadcasted_iota(jnp.int32, (1, p), 1)
    wpos = lin - (lin // w2) * w2
    zero = jnp.zeros((), bf16)

    @pl.when(i == 0)
    def _init():
        sum_s[...] = jnp.zeros_like(sum_s)
        ssq_s[...] = jnp.zeros_like(ssq_s)

    @pl.when(i < nsteps)
    def _phase1():
        acc_sum = sum_s[...]
        acc_ssq = ssq_s[...]
        for bb in range(nb):
            t1_s[bb] = jnp.dot(x_ref[bb], aw_ref[...],
                               preferred_element_type=f32)

            for t in range(h2 // 2):
                halves = []
                for hh in (2 * t, 2 * t + 1):
                    i0, i1, a0, a1 = h_taps[hh]
                    r = a0 * t1_s[bb, i0 * cin_p:(i0 + 1) * cin_p, :]
                    if a1 != 0.0:
                        r = r + a1 * t1_s[bb, i1 * cin_p:(i1 + 1) * cin_p, :]
                    halves.append(r)
                img_s[bb, c2:c2 + cin_p, 2 * t * w2:(2 * t + 2) * w2] = (
                    jnp.concatenate(halves, axis=1).astype(bf16))

            img_s[bb, c2 + cin_p:2 * c2, :] = s_ref[bb].astype(bf16)

            c_blk = img_s[bb, c2:2 * c2, :]
            img_s[bb, 0:c2, :] = jnp.where(wpos >= 1,
                                           jnp.roll(c_blk, 1, axis=1), zero)
            img_s[bb, 2 * c2:3 * c2, :] = jnp.where(
                wpos <= w2 - 2, jnp.roll(c_blk, -1, axis=1), zero)

            img = img_s[bb]
            for ky in range(3):
                wk = w_ref[ky].reshape(kw * c2, cout).astype(bf16)
                rs_s[bb, ky * cout:(ky + 1) * cout, :] = jax.lax.dot_general(
                    wk, img, (((0,), (0,)), ((), ())),
                    preferred_element_type=f32).astype(bf16)

            mid = rs_s[bb, cout:2 * cout, :].astype(f32)
            top = jnp.where(lin >= w2,
                            jnp.roll(rs_s[bb, 0:cout, :], w2, axis=1),
                            zero).astype(f32)
            bot = jnp.where(lin < p - w2,
                            jnp.roll(rs_s[bb, 2 * cout:3 * cout, :], -w2,
                                     axis=1), zero).astype(f32)
            y = jnp.maximum(mid + top + bot, 0.0)

            acc_sum = acc_sum + jnp.sum(y, axis=1, keepdims=True)
            acc_ssq = acc_ssq + jnp.sum(y * y, axis=1, keepdims=True)
            y_s[i * nb + bb] = y.astype(bf16)
        sum_s[...] = acc_sum
        ssq_s[...] = acc_ssq

    @pl.when(i >= nsteps)
    def _phase2():
        mean = sum_s[...] * (1.0 / cnt)
        var = jnp.maximum(ssq_s[...] * (1.0 / cnt) - mean * mean, 0.0)
        scale = g_ref[...] * jax.lax.rsqrt(var + eps)
        shift = b_ref[...] - mean * scale
        j = i - nsteps
        for bb in range(nb):
            o_ref[bb] = y_s[j * nb + bb].astype(f32) * scale + shift


def kernel(x_nchw, skip_nchw, w_hwio, gamma, beta):
    n, cin, h, w = x_nchw.shape
    _, cin_s, h2, w2 = skip_nchw.shape
    kh, kw, cin2, cout = w_hwio.shape
    assert (h2, w2) == (2 * h, 2 * w) and cin_s == cin and cin2 == 2 * cin
    assert kh == 3 and kw == 3
    p = h2 * w2
    cin_p = _round_up(cin, 8)
    c2 = 2 * cin_p
    f32 = jnp.float32
    bf16 = jnp.bfloat16

    aw = jnp.asarray(_width_matrix_np(w, w2), dtype=bf16)
    h_taps = _height_taps(h, h2)

    xp = x_nchw
    sp = skip_nchw
    wq = w_hwio
    if cin_p != cin:
        xp = jnp.pad(xp, ((0, 0), (0, cin_p - cin), (0, 0), (0, 0)))
        sp = jnp.pad(sp, ((0, 0), (0, cin_p - cin), (0, 0), (0, 0)))
        wq = jnp.concatenate(
            [jnp.pad(w_hwio[:, :, :cin, :],
                     ((0, 0), (0, 0), (0, cin_p - cin), (0, 0))),
             jnp.pad(w_hwio[:, :, cin:, :],
                     ((0, 0), (0, 0), (0, cin_p - cin), (0, 0)))], axis=2)
    x2d = jnp.transpose(xp, (0, 2, 1, 3)).reshape(n, h * cin_p, w)
    x2d = x2d.astype(bf16)
    s_flat = sp.reshape(n, cin_p, p)

    nb = 2 if n % 2 == 0 else 1
    nsteps = n // nb
    body = functools.partial(_fused_kernel, h_taps=h_taps, kw=kw, h2=h2,
                             w2=w2, cin_p=cin_p, cout=cout, nb=nb,
                             nsteps=nsteps, cnt=float(n * p), eps=_EPS)

    last = nsteps - 1
    out_flat = pl.pallas_call(
        body,
        out_shape=jax.ShapeDtypeStruct((n, cout, p), f32),
        grid=(2 * nsteps,),
        in_specs=[
            pl.BlockSpec((w, w2), lambda i: (0, 0)),
            pl.BlockSpec((kh, kw, c2, cout), lambda i: (0, 0, 0, 0)),
            pl.BlockSpec((cout, 1), lambda i: (0, 0)),
            pl.BlockSpec((cout, 1), lambda i: (0, 0)),
            pl.BlockSpec((nb, h * cin_p, w),
                         lambda i: (jnp.minimum(i, last), 0, 0)),
            pl.BlockSpec((nb, cin_p, p),
                         lambda i: (jnp.minimum(i, last), 0, 0)),
        ],
        out_specs=pl.BlockSpec(
            (nb, cout, p),
            lambda i: (jnp.maximum(i - nsteps, 0), 0, 0)),
        scratch_shapes=[
            pltpu.VMEM((nb, h * cin_p, w2), f32),
            pltpu.VMEM((nb, 3 * c2, p), bf16),
            pltpu.VMEM((nb, kh * cout, p), bf16),
            pltpu.VMEM((n, cout, p), bf16),
            pltpu.VMEM((cout, 1), f32),
            pltpu.VMEM((cout, 1), f32),
        ],
        compiler_params=pltpu.CompilerParams(
            dimension_semantics=("arbitrary",)),
    )(aw, wq, gamma.reshape(cout, 1).astype(f32),
      beta.reshape(cout, 1).astype(f32), x2d, s_flat)

    return out_flat.reshape(n, cout, h2, w2).astype(x_nchw.dtype)


# speedup vs baseline: 1.4384x; 1.0056x over previous
import functools
import math

import numpy as np
import jax
import jax.numpy as jnp
from jax.experimental import pallas as pl
from jax.experimental.pallas import tpu as pltpu

_EPS = 1e-5


def _round_up(v, m):
    return ((v + m - 1) // m) * m


def _width_matrix_np(n_in, n_out):
    if n_in == 1:
        return np.ones((1, n_out), np.float32)
    src = np.arange(n_out, dtype=np.float64) * (n_in - 1) / (n_out - 1)
    grid = np.arange(n_in, dtype=np.float64)
    m = np.maximum(0.0, 1.0 - np.abs(src[None, :] - grid[:, None]))
    return m.astype(np.float32)


def _height_taps(n_in, n_out):
    taps = []
    for dst in range(n_out):
        if n_in == 1:
            taps.append((0, 0, 1.0, 0.0))
            continue
        src = dst * (n_in - 1) / (n_out - 1)
        i0 = min(int(math.floor(src)), n_in - 1)
        i1 = min(i0 + 1, n_in - 1)
        frac = src - i0
        taps.append((i0, i1, 1.0 - frac, float(frac)))
    return tuple(taps)


def _fused_kernel(aw_ref, w_ref, g_ref, b_ref, x_ref, s_ref, o_ref,
                  t1_s, img_s, rs_s, y_s, sum_s, ssq_s,
                  *, h_taps, kw, h2, w2, cin_p, cout, nb, nsteps, cnt, eps):
    p = h2 * w2
    c2 = 2 * cin_p
    f32 = jnp.float32
    bf16 = jnp.bfloat16
    i = pl.program_id(0)

    lin = jax.lax.broadcasted_iota(jnp.int32, (1, p), 1)
    wpos = lin - (lin // w2) * w2
    zero = jnp.zeros((), bf16)

    @pl.when(i == 0)
    def _init():
        sum_s[...] = jnp.zeros_like(sum_s)
        ssq_s[...] = jnp.zeros_like(ssq_s)

    @pl.when(i < nsteps)
    def _phase1():
        acc_sum = sum_s[...]
        acc_ssq = ssq_s[...]
        for bb in range(nb):
            t1_s[bb] = jnp.dot(x_ref[bb], aw_ref[...],
                               preferred_element_type=f32)

            for t in range(h2 // 2):
                halves = []
                for hh in (2 * t, 2 * t + 1):
                    i0, i1, a0, a1 = h_taps[hh]
                    r = a0 * t1_s[bb, i0 * cin_p:(i0 + 1) * cin_p, :]
                    if a1 != 0.0:
                        r = r + a1 * t1_s[bb, i1 * cin_p:(i1 + 1) * cin_p, :]
                    halves.append(r)
                img_s[bb, c2:c2 + cin_p, 2 * t * w2:(2 * t + 2) * w2] = (
                    jnp.concatenate(halves, axis=1).astype(bf16))

            img_s[bb, c2 + cin_p:2 * c2, :] = s_ref[bb].astype(bf16)

            c_blk = img_s[bb, c2:2 * c2, :]
            img_s[bb, 0:c2, :] = jnp.where(wpos >= 1,
                                           jnp.roll(c_blk, 1, axis=1), zero)
            img_s[bb, 2 * c2:3 * c2, :] = jnp.where(
                wpos <= w2 - 2, jnp.roll(c_blk, -1, axis=1), zero)

            img = img_s[bb]
            for ki, ky in enumerate((0, 2)):
                wk = w_ref[ky].reshape(kw * c2, cout).astype(bf16)
                rs_s[bb, ki * cout:(ki + 1) * cout, :] = jax.lax.dot_general(
                    wk, img, (((0,), (0,)), ((), ())),
                    preferred_element_type=f32).astype(bf16)
            wk = w_ref[1].reshape(kw * c2, cout).astype(bf16)
            mid = jax.lax.dot_general(wk, img, (((0,), (0,)), ((), ())),
                                      preferred_element_type=f32)

            top = jnp.where(lin >= w2,
                            jnp.roll(rs_s[bb, 0:cout, :], w2, axis=1),
                            zero).astype(f32)
            bot = jnp.where(lin < p - w2,
                            jnp.roll(rs_s[bb, cout:2 * cout, :], -w2,
                                     axis=1), zero).astype(f32)
            y = jnp.maximum(mid + top + bot, 0.0)

            acc_sum = acc_sum + jnp.sum(y, axis=1, keepdims=True)
            acc_ssq = acc_ssq + jnp.sum(y * y, axis=1, keepdims=True)
            y_s[i * nb + bb] = y.astype(bf16)
        sum_s[...] = acc_sum
        ssq_s[...] = acc_ssq

    @pl.when(i >= nsteps)
    def _phase2():
        mean = sum_s[...] * (1.0 / cnt)
        var = jnp.maximum(ssq_s[...] * (1.0 / cnt) - mean * mean, 0.0)
        scale = g_ref[...] * jax.lax.rsqrt(var + eps)
        shift = b_ref[...] - mean * scale
        j = i - nsteps
        for bb in range(nb):
            o_ref[bb] = y_s[j * nb + bb].astype(f32) * scale + shift


def kernel(x_nchw, skip_nchw, w_hwio, gamma, beta):
    n, cin, h, w = x_nchw.shape
    _, cin_s, h2, w2 = skip_nchw.shape
    kh, kw, cin2, cout = w_hwio.shape
    assert (h2, w2) == (2 * h, 2 * w) and cin_s == cin and cin2 == 2 * cin
    assert kh == 3 and kw == 3
    p = h2 * w2
    cin_p = _round_up(cin, 8)
    c2 = 2 * cin_p
    f32 = jnp.float32
    bf16 = jnp.bfloat16

    aw = jnp.asarray(_width_matrix_np(w, w2), dtype=bf16)
    h_taps = _height_taps(h, h2)

    xp = x_nchw
    sp = skip_nchw
    wq = w_hwio
    if cin_p != cin:
        xp = jnp.pad(xp, ((0, 0), (0, cin_p - cin), (0, 0), (0, 0)))
        sp = jnp.pad(sp, ((0, 0), (0, cin_p - cin), (0, 0), (0, 0)))
        wq = jnp.concatenate(
            [jnp.pad(w_hwio[:, :, :cin, :],
                     ((0, 0), (0, 0), (0, cin_p - cin), (0, 0))),
             jnp.pad(w_hwio[:, :, cin:, :],
                     ((0, 0), (0, 0), (0, cin_p - cin), (0, 0)))], axis=2)
    x2d = jnp.transpose(xp, (0, 2, 1, 3)).reshape(n, h * cin_p, w)
    x2d = x2d.astype(bf16)
    s_flat = sp.reshape(n, cin_p, p)

    nb = 2 if n % 2 == 0 else 1
    nsteps = n // nb
    body = functools.partial(_fused_kernel, h_taps=h_taps, kw=kw, h2=h2,
                             w2=w2, cin_p=cin_p, cout=cout, nb=nb,
                             nsteps=nsteps, cnt=float(n * p), eps=_EPS)

    last = nsteps - 1
    out_flat = pl.pallas_call(
        body,
        out_shape=jax.ShapeDtypeStruct((n, cout, p), f32),
        grid=(2 * nsteps,),
        in_specs=[
            pl.BlockSpec((w, w2), lambda i: (0, 0)),
            pl.BlockSpec((kh, kw, c2, cout), lambda i: (0, 0, 0, 0)),
            pl.BlockSpec((cout, 1), lambda i: (0, 0)),
            pl.BlockSpec((cout, 1), lambda i: (0, 0)),
            pl.BlockSpec((nb, h * cin_p, w),
                         lambda i: (jnp.minimum(i, last), 0, 0)),
            pl.BlockSpec((nb, cin_p, p),
                         lambda i: (jnp.minimum(i, last), 0, 0)),
        ],
        out_specs=pl.BlockSpec(
            (nb, cout, p),
            lambda i: (jnp.maximum(i - nsteps, 0), 0, 0)),
        scratch_shapes=[
            pltpu.VMEM((nb, h * cin_p, w2), f32),
            pltpu.VMEM((nb, 3 * c2, p), bf16),
            pltpu.VMEM((nb, 2 * cout, p), bf16),
            pltpu.VMEM((n, cout, p), bf16),
            pltpu.VMEM((cout, 1), f32),
            pltpu.VMEM((cout, 1), f32),
        ],
        compiler_params=pltpu.CompilerParams(
            dimension_semantics=("arbitrary",)),
    )(aw, wq, gamma.reshape(cout, 1).astype(f32),
      beta.reshape(cout, 1).astype(f32), x2d, s_flat)

    return out_flat.reshape(n, cout, h2, w2).astype(x_nchw.dtype)
